# baseline (speedup 1.0000x reference)
import functools
import math

import numpy as np
import jax
import jax.numpy as jnp
from jax import lax
from jax.experimental import pallas as pl
from jax.experimental.pallas import tpu as pltpu
from jax.experimental.pallas import tpu_sc as plsc

NLAT = 181
NLON = 360
NCH = 128
NHEADS = 4
CPH = NCH // NHEADS
WPAD = NLON + 24
WOUT = 368
M_TC = 16
POLAR_HO = tuple(range(M_TC + 1)) + tuple(range(NLAT - 1 - M_TC, NLAT))
POLAR_BASE = tuple(min(max(ho - 1, 0), NLAT - 3) for ho in POLAR_HO)
NPOL = len(POLAR_HO)
MID_LO = M_TC + 1
MID_HI = NLAT - 2 - M_TC
NMID = NLAT - NPOL
NTASK = NMID * NHEADS
NWORKER = 32
TASK_ITERS = -(-NTASK // NWORKER)


def _band_tables():
    n = NLAT
    N = n - 1
    j = np.arange(n)
    theta = np.pi * j / N
    K = N // 2
    k = np.arange(1, K + 1)
    b = np.where(2 * k == N, 1.0, 2.0)
    s = np.sum((b / (4.0 * k ** 2 - 1.0))[None, :] * np.cos(2.0 * np.outer(theta, k)), axis=1)
    c = np.where((j == 0) | (j == N), 1.0, 2.0)
    wgl = c / N * (1.0 - s)
    quad = 2.0 * np.pi * wgl / NLON

    cutoff = np.pi / float(N) * (1.0 + 1e-5)
    phi = 2.0 * np.pi * np.arange(NLON) / NLON
    ct = np.cos(theta)[:, None]
    st = np.sin(theta)[:, None]
    cphi = np.cos(phi)[None, :]
    T = np.zeros((n, 3), dtype=np.int32)
    for ho in range(n):
        cosd = np.cos(theta[ho]) * ct + np.sin(theta[ho]) * st * cphi
        dist = np.arccos(np.clip(cosd, -1.0, 1.0))
        mask = dist <= cutoff
        for d in range(3):
            r = ho + d - 1
            if 0 <= r < n:
                T[ho, d] = int(mask[r].sum())
    l0 = (T[1:n - 1, 1] - 1) // 2
    mm = np.minimum(np.arange(1, n - 1), N - np.arange(1, n - 1))
    lmax = int(l0.max())
    mmax = [int(mm[l0 >= j].max()) for j in range(1, lmax + 1)]
    rows = np.clip(np.arange(n)[:, None] + np.arange(-1, 2)[None, :], 0, N)
    q_tab = np.broadcast_to(quad[rows][:, :, None], (n, 3, 16)).astype(np.float32)

    dd = (np.arange(NLON)[:, None] - np.arange(NLON)[None, :]) % NLON
    dmin = np.minimum(dd, NLON - dd)
    madd = np.full((NPOL, 3, NLON, NLON), -3e38, dtype=np.float32)
    for i, (ho, base) in enumerate(zip(POLAR_HO, POLAR_BASE)):
        for d in range(3):
            r = base + d
            cnt = T[ho, r - ho + 1] if abs(r - ho) <= 1 else 0
            if cnt == 0:
                continue
            lw = 180 if cnt == NLON else (cnt - 1) // 2
            madd[i, d][dmin <= lw] = np.log(quad[r])
    return tuple(mmax), np.ascontiguousarray(q_tab), madd


_MMAX, _Q_TAB, _POLAR_MASK = _band_tables()


def _qkv_body(x_ref, w_ref, bq_ref, bk_ref, bv_ref, q_ref, k_ref, v_ref):
    x = x_ref[:, 0, 0, :]
    big = jnp.dot(w_ref[...], x, preferred_element_type=jnp.float32)
    q = big[:NCH] + bq_ref[...]
    k = big[NCH:2 * NCH] + bk_ref[...]
    v = big[2 * NCH:] + bv_ref[...]
    for ref, val in ((q_ref, q), (k_ref, k), (v_ref, v)):
        ref[0, :, :NLON] = val
        ref[0, :, NLON:] = val[:, :WPAD - NLON]


def _qkv_call(x, w_all, bq, bk, bv):
    row = jax.ShapeDtypeStruct((NLAT, NCH, WPAD), jnp.float32)
    return pl.pallas_call(
        _qkv_body,
        grid=(NLAT,),
        in_specs=[
            pl.BlockSpec((NCH, 1, 1, NLON), lambda i: (0, i, 0, 0)),
            pl.BlockSpec((3 * NCH, NCH), lambda i: (0, 0)),
            pl.BlockSpec((NCH, 1), lambda i: (0, 0)),
            pl.BlockSpec((NCH, 1), lambda i: (0, 0)),
            pl.BlockSpec((NCH, 1), lambda i: (0, 0)),
        ],
        out_specs=[
            pl.BlockSpec((1, NCH, WPAD), lambda i: (i, 0, 0)),
            pl.BlockSpec((1, NCH, WPAD), lambda i: (i, 0, 0)),
            pl.BlockSpec((1, NCH, WPAD), lambda i: (i, 0, 0)),
        ],
        out_shape=[row, row, row],
    )(x, w_all, bq, bk, bv)


def _proj_body(amid_ref, apol_ref, w_ref, b_ref, o_ref):
    i = pl.program_id(0)
    polar = (i < MID_LO) | (i > MID_HI)
    a = jnp.where(polar, apol_ref[0][:, :NLON], amid_ref[0][:, :NLON])
    o_ref[:, 0, 0, :] = jnp.dot(w_ref[...], a, preferred_element_type=jnp.float32) + b_ref[...]


def _proj_call(amid, apol, w, b):
    def mid_map(i):
        return (jnp.clip(i - MID_LO, 0, NMID - 1), 0, 0)

    def pol_map(i):
        return (jnp.clip(jnp.where(i < MID_LO, i, i - NMID), 0, NPOL - 1), 0, 0)

    return pl.pallas_call(
        _proj_body,
        grid=(NLAT,),
        in_specs=[
            pl.BlockSpec((1, NCH, WOUT), mid_map),
            pl.BlockSpec((1, NCH, WOUT), pol_map),
            pl.BlockSpec((NCH, NCH), lambda i: (0, 0)),
            pl.BlockSpec((NCH, 1), lambda i: (0, 0)),
        ],
        out_specs=pl.BlockSpec((NCH, 1, 1, NLON), lambda i: (0, i, 0, 0)),
        out_shape=jax.ShapeDtypeStruct((NCH, NLAT, 1, NLON), jnp.float32),
    )(amid, apol, w, b)


def _polar_body(q_ref, k0_ref, k1_ref, k2_ref, v0_ref, v1_ref, v2_ref,
                m_ref, o_ref):
    o_ref[0, :, NLON:] = jnp.zeros((NCH, WOUT - NLON), dtype=jnp.float32)
    k_refs = (k0_ref, k1_ref, k2_ref)
    v_refs = (v0_ref, v1_ref, v2_ref)
    for h in range(NHEADS):
        qh = q_ref[0, h * CPH:(h + 1) * CPH, :NLON]
        s = []
        for d in range(3):
            kd = k_refs[d][0, h * CPH:(h + 1) * CPH, :NLON]
            sd = lax.dot_general(qh, kd, (((0,), (0,)), ((), ())),
                                 preferred_element_type=jnp.float32)
            s.append(sd + m_ref[0, d])
        m = jnp.max(s[0], axis=1)
        for d in range(1, 3):
            m = jnp.maximum(m, jnp.max(s[d], axis=1))
        acc = None
        denom = None
        for d in range(3):
            e = jnp.exp(s[d] - m[:, None])
            vd = v_refs[d][0, h * CPH:(h + 1) * CPH, :NLON]
            term = lax.dot_general(vd, e, (((1,), (1,)), ((), ())),
                                   preferred_element_type=jnp.float32)
            dsum = jnp.sum(e, axis=1)
            acc = term if acc is None else acc + term
            denom = dsum if denom is None else denom + dsum
        o_ref[0, h * CPH:(h + 1) * CPH, :NLON] = acc / denom[None, :]


def _polar_call(qp, kp, vp, mask):
    def ho_of(i):
        return jnp.where(i <= M_TC, i, i + NLAT - NPOL)

    def base_spec(d):
        def imap(i):
            base = jnp.clip(ho_of(i) - 1, 0, NLAT - 3)
            return (base + d, 0, 0)
        return pl.BlockSpec((1, NCH, WPAD), imap)

    return pl.pallas_call(
        _polar_body,
        grid=(NPOL,),
        in_specs=[
            pl.BlockSpec((1, NCH, WPAD), lambda i: (ho_of(i), 0, 0)),
            base_spec(0), base_spec(1), base_spec(2),
            base_spec(0), base_spec(1), base_spec(2),
            pl.BlockSpec((1, 3, NLON, NLON), lambda i: (i, 0, 0, 0)),
        ],
        out_specs=pl.BlockSpec((1, NCH, WOUT), lambda i: (i, 0, 0)),
        out_shape=jax.ShapeDtypeStruct((NPOL, NCH, WOUT), jnp.float32),
    )(qp, kp, kp, kp, vp, vp, vp, mask)


_GDN = lax.GatherDimensionNumbers(
    offset_dims=(), collapsed_slice_dims=(0,), start_index_map=(0,))


def _vperm(x, p):
    return lax.gather(x, p[:, None], _GDN, (1,),
                      mode=lax.GatherScatterMode.PROMISE_IN_BOUNDS)

def _att_task(ho, h, q_hbm, k_hbm, v_hbm, qd_hbm, out_hbm,
              qv, ksl, vsl, ov, cbuf, qwv, sem):
    copies = [
        pltpu.async_copy(q_hbm.at[ho, h], qv, sem),
        pltpu.async_copy(qd_hbm.at[ho], qwv, sem),
        pltpu.async_copy(k_hbm.at[pl.ds(ho - 1, 3), h], ksl, sem),
        pltpu.async_copy(v_hbm.at[pl.ds(ho - 1, 3), h], vsl, sem),
    ]
    krs = (ksl.at[0], ksl.at[1], ksl.at[2])
    vrs = (vsl.at[0], vsl.at[1], vsl.at[2])
    for cp in copies:
        cp.wait()

    mpole = jnp.minimum(ho, NLAT - 1 - ho)
    l0 = jnp.int32(0)
    for bp in _MMAX:
        l0 = l0 + jnp.where(mpole <= bp, 1, 0).astype(jnp.int32)
    t0 = 2 * l0 + 1

    lanes = lax.iota(jnp.int32, 16)

    def chunk_body(i, _):
        w0 = 16 * i
        qc = [qv[c, pl.ds(w0, 16)] for c in range(CPH)]

        def corr_aligned(kr):
            corr = None
            for c in range(CPH):
                term = qc[c] * kr[c, pl.ds(w0, 16)]
                corr = term if corr is None else corr + term
            return corr

        c0 = corr_aligned(krs[0])
        c2 = corr_aligned(krs[2])
        m = jnp.maximum(c0, c2)

        def p1(j, m):
            idx = lax.rem(w0 + j - l0 + 720, NLON)
            ab = (idx // 16) * 16
            r = idx - ab
            lm = lanes >= r
            p = lax.rem(lanes + r, 16)
            corr = None
            for c in range(CPH):
                a = krs[1][c, pl.ds(ab, 16)]
                b = krs[1][c, pl.ds(ab + 16, 16)]
                win = _vperm(jnp.where(lm, a, b), p)
                term = qc[c] * win
                corr = term if corr is None else corr + term
            cbuf[j % 8, pl.ds((j // 8) * 16, 16)] = corr
            return jnp.maximum(m, corr)

        m = lax.fori_loop(0, t0, p1, m)

        e0 = jnp.exp(c0 - m) * qwv[0]
        e2 = jnp.exp(c2 - m) * qwv[2]
        denom = e0 + e2
        accs = tuple(e0 * vrs[0][c, pl.ds(w0, 16)] + e2 * vrs[2][c, pl.ds(w0, 16)]
                     for c in range(CPH))

        def p2(j, carry):
            dnm = carry[0]
            acc = carry[1:]
            idx = lax.rem(w0 + j - l0 + 720, NLON)
            ab = (idx // 16) * 16
            r = idx - ab
            lm = lanes >= r
            p = lax.rem(lanes + r, 16)
            e = jnp.exp(cbuf[j % 8, pl.ds((j // 8) * 16, 16)] - m) * qwv[1]
            acc = tuple(
                acc[c] + e * _vperm(
                    jnp.where(lm, vrs[1][c, pl.ds(ab, 16)],
                              vrs[1][c, pl.ds(ab + 16, 16)]), p)
                for c in range(CPH))
            return (dnm + e,) + acc

        denom, *accs = lax.fori_loop(0, t0, p2, (denom,) + accs)

        rec = 1.0 / denom
        for c in range(CPH):
            ov[c, pl.ds(w0, 16)] = accs[c] * rec
        return 0

    lax.fori_loop(0, WOUT // 16, chunk_body, 0)
    pltpu.async_copy(ov, out_hbm.at[ho - MID_LO, h], sem).wait()


def _make_att_kernel():
    info = plsc.get_sparse_core_info()
    nc, ns = info.num_cores, info.num_subcores
    mesh = plsc.VectorSubcoreMesh(core_axis_name="c", subcore_axis_name="s")
    slab = pltpu.VMEM((3, CPH, WPAD), jnp.float32)

    @functools.partial(
        pl.kernel,
        mesh=mesh,
        out_type=jax.ShapeDtypeStruct((NMID, NHEADS, CPH, WOUT), jnp.float32),
        scratch_types=[
            pltpu.VMEM((CPH, WPAD), jnp.float32),
            slab, slab,
            pltpu.VMEM((CPH, WOUT), jnp.float32),
            pltpu.VMEM((8, (3 * NLON // 8) * 16), jnp.float32),
            pltpu.VMEM((3, 16), jnp.float32),
            pltpu.SemaphoreType.DMA,
        ],
    )
    def att(q_hbm, k_hbm, v_hbm, qd_hbm, out_hbm,
            qv, ksl, vsl, ov, cbuf, qwv, sem):
        wid = lax.axis_index("s") * nc + lax.axis_index("c")

        def task_body(t, _):
            task = NWORKER * t + jnp.where(t % 2 == 0, wid, NWORKER - 1 - wid)

            @pl.when(task < NTASK)
            def _():
                s = task // NHEADS
                h = task % NHEADS
                ho = jnp.where(s % 2 == 0, MID_LO + s // 2, MID_HI - s // 2)
                _att_task(ho, h, q_hbm, k_hbm, v_hbm, qd_hbm, out_hbm,
                          qv, ksl, vsl, ov, cbuf, qwv, sem)

            return 0

        lax.fori_loop(0, TASK_ITERS, task_body, 0)

    return att


def kernel(query, q_weights, k_weights, v_weights, proj_weights,
           q_bias, k_bias, v_bias, proj_bias):
    scale = 1.0 / math.sqrt(NCH)
    x3d = query.reshape(NCH, NLAT, 1, NLON)
    w_all = jnp.concatenate([q_weights * scale, k_weights, v_weights], axis=0)
    qp, kp, vp = _qkv_call(x3d, w_all, q_bias.reshape(NCH, 1),
                           k_bias.reshape(NCH, 1), v_bias.reshape(NCH, 1))
    qh = qp.reshape(NLAT, NHEADS, CPH, WPAD)
    kh = kp.reshape(NLAT, NHEADS, CPH, WPAD)
    vh = vp.reshape(NLAT, NHEADS, CPH, WPAD)
    att_mid = _make_att_kernel()(qh, kh, vh, jnp.asarray(_Q_TAB))

    att_pol = _polar_call(qp, kp, vp, jnp.asarray(_POLAR_MASK))

    out = _proj_call(att_mid.reshape(NMID, NCH, WOUT), att_pol,
                     proj_weights, proj_bias.reshape(NCH, 1))
    return out.reshape(1, NCH, NLAT, NLON)


# speedup vs baseline: 1.0213x; 1.0213x over previous
import functools
import math

import numpy as np
import jax
import jax.numpy as jnp
from jax import lax
from jax.experimental import pallas as pl
from jax.experimental.pallas import tpu as pltpu
from jax.experimental.pallas import tpu_sc as plsc

NLAT = 181
NLON = 360
NCH = 128
NHEADS = 4
CPH = NCH // NHEADS
WPAD = NLON + 24
WOUT = 368
M_TC = 14
POLAR_HO = tuple(range(M_TC + 1)) + tuple(range(NLAT - 1 - M_TC, NLAT))
POLAR_BASE = tuple(min(max(ho - 1, 0), NLAT - 3) for ho in POLAR_HO)
NPOL = len(POLAR_HO)
MID_LO = M_TC + 1
MID_HI = NLAT - 2 - M_TC
NMID = NLAT - NPOL
NTASK = NMID * NHEADS
NWORKER = 32
TASK_ITERS = -(-NTASK // NWORKER)


def _band_tables():
    n = NLAT
    N = n - 1
    j = np.arange(n)
    theta = np.pi * j / N
    K = N // 2
    k = np.arange(1, K + 1)
    b = np.where(2 * k == N, 1.0, 2.0)
    s = np.sum((b / (4.0 * k ** 2 - 1.0))[None, :] * np.cos(2.0 * np.outer(theta, k)), axis=1)
    c = np.where((j == 0) | (j == N), 1.0, 2.0)
    wgl = c / N * (1.0 - s)
    quad = 2.0 * np.pi * wgl / NLON

    cutoff = np.pi / float(N) * (1.0 + 1e-5)
    phi = 2.0 * np.pi * np.arange(NLON) / NLON
    ct = np.cos(theta)[:, None]
    st = np.sin(theta)[:, None]
    cphi = np.cos(phi)[None, :]
    T = np.zeros((n, 3), dtype=np.int32)
    for ho in range(n):
        cosd = np.cos(theta[ho]) * ct + np.sin(theta[ho]) * st * cphi
        dist = np.arccos(np.clip(cosd, -1.0, 1.0))
        mask = dist <= cutoff
        for d in range(3):
            r = ho + d - 1
            if 0 <= r < n:
                T[ho, d] = int(mask[r].sum())
    l0 = (T[1:n - 1, 1] - 1) // 2
    mm = np.minimum(np.arange(1, n - 1), N - np.arange(1, n - 1))
    lmax = int(l0.max())
    mmax = [int(mm[l0 >= j].max()) for j in range(1, lmax + 1)]
    rows = np.clip(np.arange(n)[:, None] + np.arange(-1, 2)[None, :], 0, N)
    q_tab = np.broadcast_to(quad[rows][:, :, None], (n, 3, 16)).astype(np.float32)

    dd = (np.arange(NLON)[:, None] - np.arange(NLON)[None, :]) % NLON
    dmin = np.minimum(dd, NLON - dd)
    madd = np.full((NPOL, 3, NLON, NLON), -3e38, dtype=np.float32)
    for i, (ho, base) in enumerate(zip(POLAR_HO, POLAR_BASE)):
        for d in range(3):
            r = base + d
            cnt = T[ho, r - ho + 1] if abs(r - ho) <= 1 else 0
            if cnt == 0:
                continue
            lw = 180 if cnt == NLON else (cnt - 1) // 2
            madd[i, d][dmin <= lw] = np.log(quad[r])
    return tuple(mmax), np.ascontiguousarray(q_tab), madd


_MMAX, _Q_TAB, _POLAR_MASK = _band_tables()


def _qkv_body(x_ref, w_ref, bq_ref, bk_ref, bv_ref, q_ref, k_ref, v_ref):
    x = x_ref[:, 0, 0, :]
    big = jnp.dot(w_ref[...], x, preferred_element_type=jnp.float32)
    q = big[:NCH] + bq_ref[...]
    k = big[NCH:2 * NCH] + bk_ref[...]
    v = big[2 * NCH:] + bv_ref[...]
    for ref, val in ((q_ref, q), (k_ref, k), (v_ref, v)):
        ref[0, :, :NLON] = val
        ref[0, :, NLON:] = val[:, :WPAD - NLON]


def _qkv_call(x, w_all, bq, bk, bv):
    row = jax.ShapeDtypeStruct((NLAT, NCH, WPAD), jnp.float32)
    return pl.pallas_call(
        _qkv_body,
        grid=(NLAT,),
        in_specs=[
            pl.BlockSpec((NCH, 1, 1, NLON), lambda i: (0, i, 0, 0)),
            pl.BlockSpec((3 * NCH, NCH), lambda i: (0, 0)),
            pl.BlockSpec((NCH, 1), lambda i: (0, 0)),
            pl.BlockSpec((NCH, 1), lambda i: (0, 0)),
            pl.BlockSpec((NCH, 1), lambda i: (0, 0)),
        ],
        out_specs=[
            pl.BlockSpec((1, NCH, WPAD), lambda i: (i, 0, 0)),
            pl.BlockSpec((1, NCH, WPAD), lambda i: (i, 0, 0)),
            pl.BlockSpec((1, NCH, WPAD), lambda i: (i, 0, 0)),
        ],
        out_shape=[row, row, row],
    )(x, w_all, bq, bk, bv)


def _proj_body(amid_ref, apol_ref, w_ref, b_ref, o_ref):
    i = pl.program_id(0)
    polar = (i < MID_LO) | (i > MID_HI)
    a = jnp.where(polar, apol_ref[0][:, :NLON], amid_ref[0][:, :NLON])
    o_ref[:, 0, 0, :] = jnp.dot(w_ref[...], a, preferred_element_type=jnp.float32) + b_ref[...]


def _proj_call(amid, apol, w, b):
    def mid_map(i):
        return (jnp.clip(i - MID_LO, 0, NMID - 1), 0, 0)

    def pol_map(i):
        return (jnp.clip(jnp.where(i < MID_LO, i, i - NMID), 0, NPOL - 1), 0, 0)

    return pl.pallas_call(
        _proj_body,
        grid=(NLAT,),
        in_specs=[
            pl.BlockSpec((1, NCH, WOUT), mid_map),
            pl.BlockSpec((1, NCH, WOUT), pol_map),
            pl.BlockSpec((NCH, NCH), lambda i: (0, 0)),
            pl.BlockSpec((NCH, 1), lambda i: (0, 0)),
        ],
        out_specs=pl.BlockSpec((NCH, 1, 1, NLON), lambda i: (0, i, 0, 0)),
        out_shape=jax.ShapeDtypeStruct((NCH, NLAT, 1, NLON), jnp.float32),
    )(amid, apol, w, b)


def _polar_body(q_ref, k0_ref, k1_ref, k2_ref, v0_ref, v1_ref, v2_ref,
                m_ref, o_ref):
    o_ref[0, :, NLON:] = jnp.zeros((NCH, WOUT - NLON), dtype=jnp.float32)
    k_refs = (k0_ref, k1_ref, k2_ref)
    v_refs = (v0_ref, v1_ref, v2_ref)
    for h in range(NHEADS):
        qh = q_ref[0, h * CPH:(h + 1) * CPH, :NLON]
        s = []
        for d in range(3):
            kd = k_refs[d][0, h * CPH:(h + 1) * CPH, :NLON]
            sd = lax.dot_general(qh, kd, (((0,), (0,)), ((), ())),
                                 preferred_element_type=jnp.float32)
            s.append(sd + m_ref[0, d])
        m = jnp.max(s[0], axis=1)
        for d in range(1, 3):
            m = jnp.maximum(m, jnp.max(s[d], axis=1))
        acc = None
        denom = None
        for d in range(3):
            e = jnp.exp(s[d] - m[:, None])
            vd = v_refs[d][0, h * CPH:(h + 1) * CPH, :NLON]
            term = lax.dot_general(vd, e, (((1,), (1,)), ((), ())),
                                   preferred_element_type=jnp.float32)
            dsum = jnp.sum(e, axis=1)
            acc = term if acc is None else acc + term
            denom = dsum if denom is None else denom + dsum
        o_ref[0, h * CPH:(h + 1) * CPH, :NLON] = acc / denom[None, :]


def _polar_call(qp, kp, vp, mask):
    def ho_of(i):
        return jnp.where(i <= M_TC, i, i + NLAT - NPOL)

    def base_spec(d):
        def imap(i):
            base = jnp.clip(ho_of(i) - 1, 0, NLAT - 3)
            return (base + d, 0, 0)
        return pl.BlockSpec((1, NCH, WPAD), imap)

    return pl.pallas_call(
        _polar_body,
        grid=(NPOL,),
        in_specs=[
            pl.BlockSpec((1, NCH, WPAD), lambda i: (ho_of(i), 0, 0)),
            base_spec(0), base_spec(1), base_spec(2),
            base_spec(0), base_spec(1), base_spec(2),
            pl.BlockSpec((1, 3, NLON, NLON), lambda i: (i, 0, 0, 0)),
        ],
        out_specs=pl.BlockSpec((1, NCH, WOUT), lambda i: (i, 0, 0)),
        out_shape=jax.ShapeDtypeStruct((NPOL, NCH, WOUT), jnp.float32),
    )(qp, kp, kp, kp, vp, vp, vp, mask)


_GDN = lax.GatherDimensionNumbers(
    offset_dims=(), collapsed_slice_dims=(0,), start_index_map=(0,))


def _vperm(x, p):
    return lax.gather(x, p[:, None], _GDN, (1,),
                      mode=lax.GatherScatterMode.PROMISE_IN_BOUNDS)

def _att_task(ho, h, q_hbm, k_hbm, v_hbm, qd_hbm, out_hbm,
              qv, ksl, vsl, ov, cbuf, qwv, sem):
    copies = [
        pltpu.async_copy(q_hbm.at[ho, h], qv, sem),
        pltpu.async_copy(qd_hbm.at[ho], qwv, sem),
        pltpu.async_copy(k_hbm.at[pl.ds(ho - 1, 3), h], ksl, sem),
        pltpu.async_copy(v_hbm.at[pl.ds(ho - 1, 3), h], vsl, sem),
    ]
    krs = (ksl.at[0], ksl.at[1], ksl.at[2])
    vrs = (vsl.at[0], vsl.at[1], vsl.at[2])
    for cp in copies:
        cp.wait()

    mpole = jnp.minimum(ho, NLAT - 1 - ho)
    l0 = jnp.int32(0)
    for bp in _MMAX:
        l0 = l0 + jnp.where(mpole <= bp, 1, 0).astype(jnp.int32)
    t0 = 2 * l0 + 1

    lanes = lax.iota(jnp.int32, 16)

    def chunk_body(i, _):
        w0 = 16 * i
        qc = [qv[c, pl.ds(w0, 16)] for c in range(CPH)]

        def corr_aligned(kr):
            corr = None
            for c in range(CPH):
                term = qc[c] * kr[c, pl.ds(w0, 16)]
                corr = term if corr is None else corr + term
            return corr

        c0 = corr_aligned(krs[0])
        c2 = corr_aligned(krs[2])
        m = jnp.maximum(c0, c2)

        def p1(j, m):
            idx = lax.rem(w0 + j - l0 + 720, NLON)
            ab = (idx // 16) * 16
            r = idx - ab
            lm = lanes >= r
            p = lax.rem(lanes + r, 16)
            corr = None
            for c in range(CPH):
                a = krs[1][c, pl.ds(ab, 16)]
                b = krs[1][c, pl.ds(ab + 16, 16)]
                win = _vperm(jnp.where(lm, a, b), p)
                term = qc[c] * win
                corr = term if corr is None else corr + term
            cbuf[j % 8, pl.ds((j // 8) * 16, 16)] = corr
            return jnp.maximum(m, corr)

        m = lax.fori_loop(0, t0, p1, m)

        e0 = jnp.exp(c0 - m) * qwv[0]
        e2 = jnp.exp(c2 - m) * qwv[2]
        denom = e0 + e2
        accs = tuple(e0 * vrs[0][c, pl.ds(w0, 16)] + e2 * vrs[2][c, pl.ds(w0, 16)]
                     for c in range(CPH))

        def p2(j, carry):
            dnm = carry[0]
            acc = carry[1:]
            idx = lax.rem(w0 + j - l0 + 720, NLON)
            ab = (idx // 16) * 16
            r = idx - ab
            lm = lanes >= r
            p = lax.rem(lanes + r, 16)
            e = jnp.exp(cbuf[j % 8, pl.ds((j // 8) * 16, 16)] - m) * qwv[1]
            acc = tuple(
                acc[c] + e * _vperm(
                    jnp.where(lm, vrs[1][c, pl.ds(ab, 16)],
                              vrs[1][c, pl.ds(ab + 16, 16)]), p)
                for c in range(CPH))
            return (dnm + e,) + acc

        denom, *accs = lax.fori_loop(0, t0, p2, (denom,) + accs)

        rec = 1.0 / denom
        for c in range(CPH):
            ov[c, pl.ds(w0, 16)] = accs[c] * rec
        return 0

    lax.fori_loop(0, WOUT // 16, chunk_body, 0)
    pltpu.async_copy(ov, out_hbm.at[ho - MID_LO, h], sem).wait()


def _make_att_kernel():
    info = plsc.get_sparse_core_info()
    nc, ns = info.num_cores, info.num_subcores
    mesh = plsc.VectorSubcoreMesh(core_axis_name="c", subcore_axis_name="s")
    slab = pltpu.VMEM((3, CPH, WPAD), jnp.float32)

    @functools.partial(
        pl.kernel,
        mesh=mesh,
        out_type=jax.ShapeDtypeStruct((NMID, NHEADS, CPH, WOUT), jnp.float32),
        scratch_types=[
            pltpu.VMEM((CPH, WPAD), jnp.float32),
            slab, slab,
            pltpu.VMEM((CPH, WOUT), jnp.float32),
            pltpu.VMEM((8, (3 * NLON // 8) * 16), jnp.float32),
            pltpu.VMEM((3, 16), jnp.float32),
            pltpu.SemaphoreType.DMA,
        ],
    )
    def att(q_hbm, k_hbm, v_hbm, qd_hbm, out_hbm,
            qv, ksl, vsl, ov, cbuf, qwv, sem):
        wid = lax.axis_index("s") * nc + lax.axis_index("c")

        def task_body(t, _):
            task = NWORKER * t + jnp.where(t % 2 == 0, wid, NWORKER - 1 - wid)

            @pl.when(task < NTASK)
            def _():
                s = task // NHEADS
                h = task % NHEADS
                ho = jnp.where(s % 2 == 0, MID_LO + s // 2, MID_HI - s // 2)
                _att_task(ho, h, q_hbm, k_hbm, v_hbm, qd_hbm, out_hbm,
                          qv, ksl, vsl, ov, cbuf, qwv, sem)

            return 0

        lax.fori_loop(0, TASK_ITERS, task_body, 0)

    return att


def kernel(query, q_weights, k_weights, v_weights, proj_weights,
           q_bias, k_bias, v_bias, proj_bias):
    scale = 1.0 / math.sqrt(NCH)
    x3d = query.reshape(NCH, NLAT, 1, NLON)
    w_all = jnp.concatenate([q_weights * scale, k_weights, v_weights], axis=0)
    qp, kp, vp = _qkv_call(x3d, w_all, q_bias.reshape(NCH, 1),
                           k_bias.reshape(NCH, 1), v_bias.reshape(NCH, 1))
    qh = qp.reshape(NLAT, NHEADS, CPH, WPAD)
    kh = kp.reshape(NLAT, NHEADS, CPH, WPAD)
    vh = vp.reshape(NLAT, NHEADS, CPH, WPAD)
    att_mid = _make_att_kernel()(qh, kh, vh, jnp.asarray(_Q_TAB))

    att_pol = _polar_call(qp, kp, vp, jnp.asarray(_POLAR_MASK))

    out = _proj_call(att_mid.reshape(NMID, NCH, WOUT), att_pol,
                     proj_weights, proj_bias.reshape(NCH, 1))
    return out.reshape(1, NCH, NLAT, NLON)
